# baseline (speedup 1.0000x reference)
import jax
import jax.numpy as jnp
from jax.experimental import pallas as pl

_IMG_SIZE = 512
_NUM_ANCHORS = 3
_NUM_CLASSES = 80
_ATTRS = 5 + _NUM_CLASSES
_ANCHORS_W = (10.0, 16.0, 33.0)
_ANCHORS_H = (13.0, 30.0, 23.0)


def _decode_body(x_ref, o_ref):
    v = x_ref[0]
    h, w = v.shape[1], v.shape[2]
    sig = jax.nn.sigmoid(v)
    ex = jnp.exp(v)
    rows = jax.lax.broadcasted_iota(jnp.int32, v.shape, 0)
    c = rows % _ATTRS
    gy = jax.lax.broadcasted_iota(jnp.int32, v.shape, 1).astype(jnp.float32)
    gx = jax.lax.broadcasted_iota(jnp.int32, v.shape, 2).astype(jnp.float32)
    stride = float(_IMG_SIZE) / float(h)
    aw = jnp.where(rows < _ATTRS, _ANCHORS_W[0],
                   jnp.where(rows < 2 * _ATTRS, _ANCHORS_W[1], _ANCHORS_W[2]))
    ah = jnp.where(rows < _ATTRS, _ANCHORS_H[0],
                   jnp.where(rows < 2 * _ATTRS, _ANCHORS_H[1], _ANCHORS_H[2]))
    res = jnp.where(
        c == 0, (sig + gx) * stride,
        jnp.where(
            c == 1, (sig + gy) * stride,
            jnp.where(c == 2, ex * aw, jnp.where(c == 3, ex * ah, sig)),
        ),
    )
    r4 = res.reshape(_NUM_ANCHORS, _ATTRS, h, w)
    o_ref[0] = jnp.transpose(r4, (0, 2, 3, 1)).reshape(_NUM_ANCHORS * h * w, _ATTRS)


def kernel(input):
    bs, c, in_h, in_w = input.shape
    hw = in_h * in_w
    out = pl.pallas_call(
        _decode_body,
        grid=(bs,),
        in_specs=[pl.BlockSpec((1, c, in_h, in_w), lambda b: (b, 0, 0, 0))],
        out_specs=pl.BlockSpec((1, _NUM_ANCHORS * hw, _ATTRS), lambda b: (b, 0, 0)),
        out_shape=jax.ShapeDtypeStruct((bs, _NUM_ANCHORS * hw, _ATTRS), jnp.float32),
    )(input)
    return out
